# baseline (speedup 1.0000x reference)
import functools

import jax
import jax.numpy as jnp
from jax.experimental import pallas as pl

_K = 16
_N = 4096
_BM = 512


def _knn_body(q_ref, pt_ref, sqq_ref, sqp_ref, out_ref):
    d = jax.lax.dot_general(
        q_ref[...], pt_ref[...],
        dimension_numbers=(((1,), (0,)), ((), ())),
        precision=jax.lax.Precision.DEFAULT,
        preferred_element_type=jnp.float32,
    )
    d = -2.0 * d
    d = d + sqq_ref[...]
    d = d + sqp_ref[...]
    iota = jax.lax.broadcasted_iota(jnp.int32, (1, _N), 1)
    for j in range(_K):
        im = jnp.argmin(d, axis=1).astype(jnp.int32)[:, None]
        out_ref[:, j:j + 1] = im
        d = jnp.where(iota == im, jnp.float32(jnp.inf), d)


@functools.partial(jax.jit, static_argnames=())
def _knn16(v0):
    xyz = jnp.pad(v0, ((0, 0), (0, 5)))
    sq = jnp.sum(v0 ** 2, axis=-1)
    sqq = sq[:, None]
    sqp = sq[None, :]
    pt = xyz.T
    grid = (_N // _BM,)
    return pl.pallas_call(
        _knn_body,
        grid=grid,
        in_specs=[
            pl.BlockSpec((_BM, 8), lambda i: (i, 0)),
            pl.BlockSpec((8, _N), lambda i: (0, 0)),
            pl.BlockSpec((_BM, 1), lambda i: (i, 0)),
            pl.BlockSpec((1, _N), lambda i: (0, 0)),
        ],
        out_specs=pl.BlockSpec((_BM, _K), lambda i: (i, 0)),
        out_shape=jax.ShapeDtypeStruct((_N, _K), jnp.int32),
    )(xyz, pt, sqq, sqp)


def kernel(F, vertices):
    del F
    return _knn16(vertices[0])
